# baseline (speedup 1.0000x reference)
import jax
import jax.numpy as jnp
from jax.experimental import pallas as pl
from jax.experimental.pallas import tpu as pltpu


def _roll(a, sh, ax):
    if sh == 1:
        lo, hi = (a[-1:], a[:-1]) if ax == 0 else (a[:, -1:], a[:, :-1])
    else:
        lo, hi = (a[1:], a[:1]) if ax == 0 else (a[:, 1:], a[:, :1])
    return jnp.concatenate([lo, hi], axis=ax)


def _infill_body(n_ref, x_ref, m_ref, o_ref):
    x = x_ref[0, 0]
    m = m_ref[0]

    def it(_, carry):
        x, m = carry
        xm = x * m
        nsum = _roll(xm, 1, 0) + _roll(xm, -1, 0) + _roll(xm, 1, 1) + _roll(xm, -1, 1)
        ncnt = _roll(m, 1, 0) + _roll(m, -1, 0) + _roll(m, 1, 1) + _roll(m, -1, 1)
        newval = nsum / jnp.maximum(ncnt, 1.0)
        xn = jnp.where(m > 0, x, jnp.where(ncnt > 0, newval, x))
        mn = jnp.maximum(m, (ncnt > 0).astype(m.dtype))
        return xn, mn

    x, m = jax.lax.fori_loop(0, n_ref[0], it, (x, m))
    o_ref[0, 0] = x


def _infill_tc(im1c, mf, n, *, interpret=False):
    B, C, H, W = im1c.shape
    return pl.pallas_call(
        _infill_body,
        grid=(B, C),
        in_specs=[
            pl.BlockSpec(memory_space=pltpu.SMEM),
            pl.BlockSpec((1, 1, H, W), lambda b, c: (b, c, 0, 0)),
            pl.BlockSpec((1, H, W), lambda b, c: (b, 0, 0)),
        ],
        out_specs=pl.BlockSpec((1, 1, H, W), lambda b, c: (b, c, 0, 0)),
        out_shape=jax.ShapeDtypeStruct((B, C, H, W), im1c.dtype),
        interpret=interpret,
    )(n, im1c, mf)


def _splat_xla(im0, flow):
    B, C, H, W = im0.shape
    xs = jnp.arange(W, dtype=im0.dtype)
    ys = jnp.arange(H, dtype=im0.dtype)
    gx = xs[None, None, :] + flow[..., 0]
    gy = ys[None, :, None] + flow[..., 1]
    x0 = jnp.floor(gx)
    y0 = jnp.floor(gy)
    b_idx = jnp.arange(B)[:, None, None]
    vals = jnp.transpose(im0, (0, 2, 3, 1)).reshape(-1, C)
    acc = jnp.zeros((B * H * W, C), dtype=im0.dtype)
    cnt = jnp.zeros((B * H * W,), dtype=im0.dtype)
    for dx in (0.0, 1.0):
        for dy in (0.0, 1.0):
            xi = x0 + dx
            yi = y0 + dy
            w = (1.0 - jnp.abs(gx - xi)) * (1.0 - jnp.abs(gy - yi))
            valid = (xi >= 0) & (xi <= W - 1) & (yi >= 0) & (yi <= H - 1)
            w = w * valid.astype(im0.dtype)
            xc = jnp.clip(xi, 0, W - 1).astype(jnp.int32)
            yc = jnp.clip(yi, 0, H - 1).astype(jnp.int32)
            flat = ((b_idx * H + yc) * W + xc).reshape(-1)
            acc = acc.at[flat].add(vals * w.reshape(-1, 1))
            cnt = cnt.at[flat].add(w.reshape(-1))
    im1 = jnp.transpose(acc.reshape(B, H, W, C), (0, 3, 1, 2))
    return im1, cnt.reshape(B, H, W)


def _backward_xla(im0, flowback):
    B, C, H, W = im0.shape
    xs = jnp.arange(W, dtype=im0.dtype)
    ys = jnp.arange(H, dtype=im0.dtype)
    gx = xs[None, None, :] + flowback[..., 0]
    gy = ys[None, :, None] + flowback[..., 1]
    x0 = jnp.floor(gx)
    y0 = jnp.floor(gy)
    valid = (gx >= 0) & (gx <= W - 1) & (gy >= 0) & (gy <= H - 1)
    out = jnp.zeros((B, C, H, W), dtype=im0.dtype)
    b_idx = jnp.arange(B)[:, None, None]
    for xi, yi in ((x0, y0), (x0 + 1.0, y0), (x0, y0 + 1.0), (x0 + 1.0, y0 + 1.0)):
        w = (1.0 - jnp.abs(gx - xi)) * (1.0 - jnp.abs(gy - yi))
        xc = jnp.clip(xi, 0, W - 1).astype(jnp.int32)
        yc = jnp.clip(yi, 0, H - 1).astype(jnp.int32)
        g = im0[b_idx, :, yc, xc]
        out = out + jnp.transpose(g, (0, 3, 1, 2)) * w[:, None, :, :]
    return out, valid


def kernel(im0, flow, flowback, infil_iterations):
    im1, cnt = _splat_xla(im0, flow)
    fill, valid = _backward_xla(im0, flowback)
    covered = cnt > 1e-6
    im1c = jnp.where(covered[:, None], im1,
                     fill * valid[:, None].astype(im0.dtype))
    mf = (covered | valid).astype(im0.dtype)
    n = jnp.asarray(infil_iterations, jnp.int32).reshape(1)
    return _infill_tc(im1c, mf, n)
